# baseline (speedup 1.0000x reference)
import functools

import jax
import jax.numpy as jnp
from jax import lax
from jax.experimental import pallas as pl
from jax.experimental.pallas import tpu as pltpu
from jax.experimental.pallas import tpu_sc as plsc

_B = 2048
_IN = 8192
_OUT = 8192
_L = 16
_NC = 2
_NS = 16
_NW = _NC * _NS
_ROWS_PER_TILE = _B // _NW
_NG = _OUT // _L


def _coef_body(w_ref, c0_ref, ca_ref, cb_ref, cab_ref):
    w = w_ref[...]
    m = jnp.max(w, axis=-1, keepdims=True)
    e = jnp.exp(w - m)
    p16 = e / jnp.sum(e, axis=-1, keepdims=True)
    p = [p16[:, k] for k in range(16)]
    c0_ref[...] = p[8] + p[9] + p[10] + p[11] + p[12] + p[13] + p[14] + p[15]
    ca_ref[...] = p[2] + p[3] + p[6] + p[7] - p[8] - p[9] - p[12] - p[13]
    cb_ref[...] = p[4] + p[5] + p[6] + p[7] - p[8] - p[9] - p[10] - p[11]
    cab_ref[...] = (p[1] - p[2] - p[4] - 2.0 * p[6] - p[7] + p[8]
                    + 2.0 * p[9] + p[11] + p[13] - p[14])


def _coefs(weights):
    blk = 512
    return pl.pallas_call(
        _coef_body,
        grid=(_OUT // blk,),
        in_specs=[pl.BlockSpec((blk, 16), lambda i: (i, 0))],
        out_specs=[pl.BlockSpec((blk,), lambda i: (i,))] * 4,
        out_shape=[jax.ShapeDtypeStruct((_OUT,), jnp.float32)] * 4,
    )(weights)


def _sc_body(x_hbm, ia_hbm, ib_hbm, c0_hbm, ca_hbm, cb_hbm, cab_hbm,
             out_hbm, row_v, out_v, ia_v, ib_v, c0_v, ca_v, cb_v, cab_v):
    c = lax.axis_index("c")
    s = lax.axis_index("s")
    wid = s * _NC + c
    pltpu.sync_copy(ia_hbm, ia_v)
    pltpu.sync_copy(ib_hbm, ib_v)
    pltpu.sync_copy(c0_hbm, c0_v)
    pltpu.sync_copy(ca_hbm, ca_v)
    pltpu.sync_copy(cb_hbm, cb_v)
    pltpu.sync_copy(cab_hbm, cab_v)
    row0 = wid * _ROWS_PER_TILE

    def row_body(r, carry):
        pltpu.sync_copy(x_hbm.at[row0 + r], row_v)

        def g_body(g, carry2):
            off = g * _L
            ia = ia_v[pl.ds(off, _L)]
            ib = ib_v[pl.ds(off, _L)]
            a = plsc.load_gather(row_v, [ia])
            b = plsc.load_gather(row_v, [ib])
            out_v[pl.ds(off, _L)] = (c0_v[pl.ds(off, _L)]
                                     + ca_v[pl.ds(off, _L)] * a
                                     + cb_v[pl.ds(off, _L)] * b
                                     + cab_v[pl.ds(off, _L)] * (a * b))
            return carry2

        lax.fori_loop(0, _NG, g_body, 0, unroll=4)
        pltpu.sync_copy(out_v, out_hbm.at[row0 + r])
        return carry

    lax.fori_loop(0, _ROWS_PER_TILE, row_body, 0)


_sc_main = functools.partial(
    pl.kernel,
    out_type=jax.ShapeDtypeStruct((_B, _OUT), jnp.float32),
    mesh=plsc.VectorSubcoreMesh(core_axis_name="c", subcore_axis_name="s"),
    compiler_params=pltpu.CompilerParams(needs_layout_passes=False),
    scratch_types=[
        pltpu.VMEM((_IN,), jnp.float32),
        pltpu.VMEM((_OUT,), jnp.float32),
        pltpu.VMEM((_OUT,), jnp.int32),
        pltpu.VMEM((_OUT,), jnp.int32),
        pltpu.VMEM((_OUT,), jnp.float32),
        pltpu.VMEM((_OUT,), jnp.float32),
        pltpu.VMEM((_OUT,), jnp.float32),
        pltpu.VMEM((_OUT,), jnp.float32),
    ],
)(_sc_body)


def kernel(x, weights, idx_a, idx_b):
    c0, ca, cb, cab = _coefs(weights)
    ia = idx_a.astype(jnp.int32)
    ib = idx_b.astype(jnp.int32)
    return _sc_main(x, ia, ib, c0, ca, cb, cab)


# speedup vs baseline: 3.2793x; 3.2793x over previous
import functools

import jax
import jax.numpy as jnp
from jax import lax
from jax.experimental import pallas as pl
from jax.experimental.pallas import tpu as pltpu
from jax.experimental.pallas import tpu_sc as plsc

_B = 2048
_IN = 8192
_OUT = 8192
_L = 16
_NC = 2
_NS = 16
_NW = _NC * _NS
_ROWS_PER_TILE = _B // _NW
_NG = _OUT // _L


def _coef_body(w_ref, c0_ref, ca_ref, cb_ref, cab_ref):
    w = w_ref[...]
    m = jnp.max(w, axis=-1, keepdims=True)
    e = jnp.exp(w - m)
    p16 = e / jnp.sum(e, axis=-1, keepdims=True)
    p = [p16[:, k] for k in range(16)]
    c0_ref[...] = p[8] + p[9] + p[10] + p[11] + p[12] + p[13] + p[14] + p[15]
    ca_ref[...] = p[2] + p[3] + p[6] + p[7] - p[8] - p[9] - p[12] - p[13]
    cb_ref[...] = p[4] + p[5] + p[6] + p[7] - p[8] - p[9] - p[10] - p[11]
    cab_ref[...] = (p[1] - p[2] - p[4] - 2.0 * p[6] - p[7] + p[8]
                    + 2.0 * p[9] + p[11] + p[13] - p[14])


def _coefs(weights):
    blk = 512
    return pl.pallas_call(
        _coef_body,
        grid=(_OUT // blk,),
        in_specs=[pl.BlockSpec((blk, 16), lambda i: (i, 0))],
        out_specs=[pl.BlockSpec((blk,), lambda i: (i,))] * 4,
        out_shape=[jax.ShapeDtypeStruct((_OUT,), jnp.float32)] * 4,
    )(weights)


def _sc_body(x_hbm, ia_hbm, ib_hbm, c0_hbm, ca_hbm, cb_hbm, cab_hbm,
             out_hbm,
             row0_v, row1_v, row2_v, row3_v,
             o0_v, o1_v, o2_v, o3_v,
             ia_v, ib_v, c0_v, ca_v, cb_v, cab_v,
             isem0, isem1, isem2, isem3, osem0, osem1, osem2, osem3):
    c = lax.axis_index("c")
    s = lax.axis_index("s")
    wid = s * _NC + c
    base = wid * _ROWS_PER_TILE
    pltpu.sync_copy(ia_hbm, ia_v)
    pltpu.sync_copy(ib_hbm, ib_v)
    pltpu.sync_copy(c0_hbm, c0_v)
    pltpu.sync_copy(ca_hbm, ca_v)
    pltpu.sync_copy(cb_hbm, cb_v)
    pltpu.sync_copy(cab_hbm, cab_v)

    rows = (row0_v, row1_v, row2_v, row3_v)
    outs = (o0_v, o1_v, o2_v, o3_v)
    isems = (isem0, isem1, isem2, isem3)
    osems = (osem0, osem1, osem2, osem3)

    for b in range(4):
        pltpu.async_copy(x_hbm.at[base + b], rows[b], isems[b])

    def _pair(i, h):
        b0, b1 = 2 * h, 2 * h + 1
        r0 = 4 * i + 2 * h
        rx0, rx1 = rows[b0], rows[b1]
        ou0, ou1 = outs[b0], outs[b1]
        pltpu.make_async_copy(x_hbm.at[base], rx0, isems[b0]).wait()
        pltpu.make_async_copy(x_hbm.at[base], rx1, isems[b1]).wait()

        @pl.when(i >= 1)
        def _():
            pltpu.make_async_copy(ou0, out_hbm.at[base], osems[b0]).wait()
            pltpu.make_async_copy(ou1, out_hbm.at[base], osems[b1]).wait()

        @plsc.parallel_loop(0, _NG, unroll=8)
        def _g(g):
            off = g * _L
            ia = ia_v[pl.ds(off, _L)]
            ib = ib_v[pl.ds(off, _L)]
            k0 = c0_v[pl.ds(off, _L)]
            ka = ca_v[pl.ds(off, _L)]
            kb = cb_v[pl.ds(off, _L)]
            kab = cab_v[pl.ds(off, _L)]
            a0 = plsc.load_gather(rx0, [ia])
            b0v = plsc.load_gather(rx0, [ib])
            a1 = plsc.load_gather(rx1, [ia])
            b1v = plsc.load_gather(rx1, [ib])
            ou0[pl.ds(off, _L)] = k0 + ka * a0 + kb * b0v + kab * (a0 * b0v)
            ou1[pl.ds(off, _L)] = k0 + ka * a1 + kb * b1v + kab * (a1 * b1v)

        pltpu.async_copy(ou0, out_hbm.at[base + r0], osems[b0])
        pltpu.async_copy(ou1, out_hbm.at[base + r0 + 1], osems[b1])

        @pl.when(i < _ROWS_PER_TILE // 4 - 1)
        def _():
            pltpu.async_copy(x_hbm.at[base + r0 + 4], rx0, isems[b0])
            pltpu.async_copy(x_hbm.at[base + r0 + 5], rx1, isems[b1])

    def _iter(i, carry):
        _pair(i, 0)
        _pair(i, 1)
        return carry

    lax.fori_loop(0, _ROWS_PER_TILE // 4, _iter, 0)

    for b in range(4):
        pltpu.make_async_copy(outs[b], out_hbm.at[base], osems[b]).wait()


_sc_main = functools.partial(
    pl.kernel,
    out_type=jax.ShapeDtypeStruct((_B, _OUT), jnp.float32),
    mesh=plsc.VectorSubcoreMesh(core_axis_name="c", subcore_axis_name="s"),
    compiler_params=pltpu.CompilerParams(needs_layout_passes=False),
    scratch_types=[
        pltpu.VMEM((_IN,), jnp.float32),
        pltpu.VMEM((_IN,), jnp.float32),
        pltpu.VMEM((_IN,), jnp.float32),
        pltpu.VMEM((_IN,), jnp.float32),
        pltpu.VMEM((_OUT,), jnp.float32),
        pltpu.VMEM((_OUT,), jnp.float32),
        pltpu.VMEM((_OUT,), jnp.float32),
        pltpu.VMEM((_OUT,), jnp.float32),
        pltpu.VMEM((_OUT,), jnp.int32),
        pltpu.VMEM((_OUT,), jnp.int32),
        pltpu.VMEM((_OUT,), jnp.float32),
        pltpu.VMEM((_OUT,), jnp.float32),
        pltpu.VMEM((_OUT,), jnp.float32),
        pltpu.VMEM((_OUT,), jnp.float32),
        pltpu.SemaphoreType.DMA,
        pltpu.SemaphoreType.DMA,
        pltpu.SemaphoreType.DMA,
        pltpu.SemaphoreType.DMA,
        pltpu.SemaphoreType.DMA,
        pltpu.SemaphoreType.DMA,
        pltpu.SemaphoreType.DMA,
        pltpu.SemaphoreType.DMA,
    ],
)(_sc_body)


def kernel(x, weights, idx_a, idx_b):
    c0, ca, cb, cab = _coefs(weights)
    ia = idx_a.astype(jnp.int32)
    ib = idx_b.astype(jnp.int32)
    return _sc_main(x, ia, ib, c0, ca, cb, cab)


# speedup vs baseline: 3.9259x; 1.1972x over previous
import functools

import jax
import jax.numpy as jnp
from jax import lax
from jax.experimental import pallas as pl
from jax.experimental.pallas import tpu as pltpu
from jax.experimental.pallas import tpu_sc as plsc

_B = 2048
_IN = 8192
_OUT = 8192
_L = 16
_NC = 2
_NS = 16
_NW = _NC * _NS
_ROWS_PER_TILE = _B // _NW
_NG = _OUT // _L
_JS = _OUT // _NS


def _sc_body(x_hbm, w_hbm, ia_hbm, ib_hbm,
             out_hbm, cof_hbm,
             row0_v, row1_v, row2_v, row3_v,
             o0_v, o1_v, o2_v, o3_v,
             ia_v, ib_v, c0_v, ca_v, cb_v, cab_v,
             w_v, st0_v, sta_v, stb_v, stab_v,
             isem0, isem1, isem2, isem3, osem0, osem1, osem2, osem3,
             iasem, ibsem):
    c = lax.axis_index("c")
    s = lax.axis_index("s")
    wid = s * _NC + c
    base = wid * _ROWS_PER_TILE

    rows = (row0_v, row1_v, row2_v, row3_v)
    outs = (o0_v, o1_v, o2_v, o3_v)
    isems = (isem0, isem1, isem2, isem3)
    osems = (osem0, osem1, osem2, osem3)

    pltpu.async_copy(ia_hbm, ia_v, iasem)
    pltpu.async_copy(ib_hbm, ib_v, ibsem)
    for b in range(4):
        pltpu.async_copy(x_hbm.at[base + b], rows[b], isems[b])

    jbase = s * _JS
    pltpu.sync_copy(w_hbm.at[pl.ds(jbase * 16, _JS * 16)], w_v)

    def _cgroup(g, carry):
        j0 = g * _L
        stride = lax.iota(jnp.int32, _L) * 16
        cols = []
        for k in range(16):
            cols.append(plsc.load_gather(w_v, [j0 * 16 + k + stride]))
        m = cols[0]
        for k in range(1, 16):
            m = jnp.maximum(m, cols[k])
        e = [jnp.exp(col - m) for col in cols]
        tot = e[0]
        for k in range(1, 16):
            tot = tot + e[k]
        inv = 1.0 / tot
        st0_v[pl.ds(j0, _L)] = (e[8] + e[9] + e[10] + e[11]
                                + e[12] + e[13] + e[14] + e[15]) * inv
        sta_v[pl.ds(j0, _L)] = (e[2] + e[3] + e[6] + e[7]
                                - e[8] - e[9] - e[12] - e[13]) * inv
        stb_v[pl.ds(j0, _L)] = (e[4] + e[5] + e[6] + e[7]
                                - e[8] - e[9] - e[10] - e[11]) * inv
        stab_v[pl.ds(j0, _L)] = (e[1] - e[2] - e[4] - 2.0 * e[6] - e[7]
                                 + e[8] + 2.0 * e[9] + e[11] + e[13]
                                 - e[14]) * inv
        return carry

    lax.fori_loop(0, _JS // _L, _cgroup, 0)

    pltpu.sync_copy(st0_v, cof_hbm.at[c, 0, pl.ds(jbase, _JS)])
    pltpu.sync_copy(sta_v, cof_hbm.at[c, 1, pl.ds(jbase, _JS)])
    pltpu.sync_copy(stb_v, cof_hbm.at[c, 2, pl.ds(jbase, _JS)])
    pltpu.sync_copy(stab_v, cof_hbm.at[c, 3, pl.ds(jbase, _JS)])
    plsc.subcore_barrier()
    pltpu.sync_copy(cof_hbm.at[c, 0], c0_v)
    pltpu.sync_copy(cof_hbm.at[c, 1], ca_v)
    pltpu.sync_copy(cof_hbm.at[c, 2], cb_v)
    pltpu.sync_copy(cof_hbm.at[c, 3], cab_v)

    pltpu.make_async_copy(ia_hbm, ia_v, iasem).wait()
    pltpu.make_async_copy(ib_hbm, ib_v, ibsem).wait()

    def _pair(i, h):
        b0, b1 = 2 * h, 2 * h + 1
        r0 = 4 * i + 2 * h
        rx0, rx1 = rows[b0], rows[b1]
        ou0, ou1 = outs[b0], outs[b1]
        pltpu.make_async_copy(x_hbm.at[base], rx0, isems[b0]).wait()
        pltpu.make_async_copy(x_hbm.at[base], rx1, isems[b1]).wait()

        @pl.when(i >= 1)
        def _():
            pltpu.make_async_copy(ou0, out_hbm.at[base], osems[b0]).wait()
            pltpu.make_async_copy(ou1, out_hbm.at[base], osems[b1]).wait()

        @plsc.parallel_loop(0, _NG, unroll=8)
        def _g(g):
            off = g * _L
            ia = ia_v[pl.ds(off, _L)]
            ib = ib_v[pl.ds(off, _L)]
            k0 = c0_v[pl.ds(off, _L)]
            ka = ca_v[pl.ds(off, _L)]
            kb = cb_v[pl.ds(off, _L)]
            kab = cab_v[pl.ds(off, _L)]
            a0 = plsc.load_gather(rx0, [ia])
            b0v = plsc.load_gather(rx0, [ib])
            a1 = plsc.load_gather(rx1, [ia])
            b1v = plsc.load_gather(rx1, [ib])
            ou0[pl.ds(off, _L)] = k0 + ka * a0 + kb * b0v + kab * (a0 * b0v)
            ou1[pl.ds(off, _L)] = k0 + ka * a1 + kb * b1v + kab * (a1 * b1v)

        pltpu.async_copy(ou0, out_hbm.at[base + r0], osems[b0])
        pltpu.async_copy(ou1, out_hbm.at[base + r0 + 1], osems[b1])

        @pl.when(i < _ROWS_PER_TILE // 4 - 1)
        def _():
            pltpu.async_copy(x_hbm.at[base + r0 + 4], rx0, isems[b0])
            pltpu.async_copy(x_hbm.at[base + r0 + 5], rx1, isems[b1])

    def _iter(i, carry):
        _pair(i, 0)
        _pair(i, 1)
        return carry

    lax.fori_loop(0, _ROWS_PER_TILE // 4, _iter, 0)

    for b in range(4):
        pltpu.make_async_copy(outs[b], out_hbm.at[base], osems[b]).wait()


_sc_main = functools.partial(
    pl.kernel,
    out_type=(jax.ShapeDtypeStruct((_B, _OUT), jnp.float32),
              jax.ShapeDtypeStruct((_NC, 4, _OUT), jnp.float32)),
    mesh=plsc.VectorSubcoreMesh(core_axis_name="c", subcore_axis_name="s"),
    compiler_params=pltpu.CompilerParams(needs_layout_passes=False),
    scratch_types=[
        pltpu.VMEM((_IN,), jnp.float32),
        pltpu.VMEM((_IN,), jnp.float32),
        pltpu.VMEM((_IN,), jnp.float32),
        pltpu.VMEM((_IN,), jnp.float32),
        pltpu.VMEM((_OUT,), jnp.float32),
        pltpu.VMEM((_OUT,), jnp.float32),
        pltpu.VMEM((_OUT,), jnp.float32),
        pltpu.VMEM((_OUT,), jnp.float32),
        pltpu.VMEM((_OUT,), jnp.int32),
        pltpu.VMEM((_OUT,), jnp.int32),
        pltpu.VMEM((_OUT,), jnp.float32),
        pltpu.VMEM((_OUT,), jnp.float32),
        pltpu.VMEM((_OUT,), jnp.float32),
        pltpu.VMEM((_OUT,), jnp.float32),
        pltpu.VMEM((_JS * 16,), jnp.float32),
        pltpu.VMEM((_JS,), jnp.float32),
        pltpu.VMEM((_JS,), jnp.float32),
        pltpu.VMEM((_JS,), jnp.float32),
        pltpu.VMEM((_JS,), jnp.float32),
        pltpu.SemaphoreType.DMA,
        pltpu.SemaphoreType.DMA,
        pltpu.SemaphoreType.DMA,
        pltpu.SemaphoreType.DMA,
        pltpu.SemaphoreType.DMA,
        pltpu.SemaphoreType.DMA,
        pltpu.SemaphoreType.DMA,
        pltpu.SemaphoreType.DMA,
        pltpu.SemaphoreType.DMA,
        pltpu.SemaphoreType.DMA,
    ],
)(_sc_body)


def kernel(x, weights, idx_a, idx_b):
    ia = idx_a.astype(jnp.int32)
    ib = idx_b.astype(jnp.int32)
    out, _ = _sc_main(x, weights.reshape(-1), ia, ib)
    return out


# speedup vs baseline: 4.2117x; 1.0728x over previous
import functools

import jax
import jax.numpy as jnp
from jax import lax
from jax.experimental import pallas as pl
from jax.experimental.pallas import tpu as pltpu
from jax.experimental.pallas import tpu_sc as plsc

_B = 2048
_IN = 8192
_OUT = 8192
_L = 16
_NC = 2
_NS = 16
_NW = _NC * _NS
_ROWS_PER_TILE = _B // _NW
_NG = _OUT // _L
_JS = _OUT // _NS


def _sc_body(x_hbm, w_hbm, ipk_hbm,
             out_hbm, cof_hbm,
             row0_v, row1_v, row2_v, row3_v,
             o0_v, o1_v, o2_v, o3_v,
             ipk_v, c0_v, ca_v, cb_v, cab_v,
             w_v, st0_v, sta_v, stb_v, stab_v,
             isem0, isem1, isem2, isem3, osem0, osem1, osem2, osem3,
             ipksem):
    c = lax.axis_index("c")
    s = lax.axis_index("s")
    wid = s * _NC + c
    base = wid * _ROWS_PER_TILE

    rows = (row0_v, row1_v, row2_v, row3_v)
    outs = (o0_v, o1_v, o2_v, o3_v)
    isems = (isem0, isem1, isem2, isem3)
    osems = (osem0, osem1, osem2, osem3)

    pltpu.async_copy(ipk_hbm, ipk_v, ipksem)
    for b in range(4):
        pltpu.async_copy(x_hbm.at[base + b], rows[b], isems[b])

    jbase = s * _JS
    pltpu.sync_copy(w_hbm.at[pl.ds(jbase * 16, _JS * 16)], w_v)

    def _cgroup(g, carry):
        j0 = g * _L
        stride = lax.iota(jnp.int32, _L) * 16
        cols = []
        for k in range(16):
            cols.append(plsc.load_gather(w_v, [j0 * 16 + k + stride]))
        m = cols[0]
        for k in range(1, 16):
            m = jnp.maximum(m, cols[k])
        e = [jnp.exp(col - m) for col in cols]
        tot = e[0]
        for k in range(1, 16):
            tot = tot + e[k]
        inv = 1.0 / tot
        st0_v[pl.ds(j0, _L)] = (e[8] + e[9] + e[10] + e[11]
                                + e[12] + e[13] + e[14] + e[15]) * inv
        sta_v[pl.ds(j0, _L)] = (e[2] + e[3] + e[6] + e[7]
                                - e[8] - e[9] - e[12] - e[13]) * inv
        stb_v[pl.ds(j0, _L)] = (e[4] + e[5] + e[6] + e[7]
                                - e[8] - e[9] - e[10] - e[11]) * inv
        stab_v[pl.ds(j0, _L)] = (e[1] - e[2] - e[4] - 2.0 * e[6] - e[7]
                                 + e[8] + 2.0 * e[9] + e[11] + e[13]
                                 - e[14]) * inv
        return carry

    lax.fori_loop(0, _JS // _L, _cgroup, 0)

    pltpu.sync_copy(st0_v, cof_hbm.at[c, 0, pl.ds(jbase, _JS)])
    pltpu.sync_copy(sta_v, cof_hbm.at[c, 1, pl.ds(jbase, _JS)])
    pltpu.sync_copy(stb_v, cof_hbm.at[c, 2, pl.ds(jbase, _JS)])
    pltpu.sync_copy(stab_v, cof_hbm.at[c, 3, pl.ds(jbase, _JS)])
    plsc.subcore_barrier()
    pltpu.sync_copy(cof_hbm.at[c, 0], c0_v)
    pltpu.sync_copy(cof_hbm.at[c, 1], ca_v)
    pltpu.sync_copy(cof_hbm.at[c, 2], cb_v)
    pltpu.sync_copy(cof_hbm.at[c, 3], cab_v)

    pltpu.make_async_copy(ipk_hbm, ipk_v, ipksem).wait()

    def _pair(i, h):
        b0, b1 = 2 * h, 2 * h + 1
        r0 = 4 * i + 2 * h
        rx0, rx1 = rows[b0], rows[b1]
        ou0, ou1 = outs[b0], outs[b1]
        pltpu.make_async_copy(x_hbm.at[base], rx0, isems[b0]).wait()
        pltpu.make_async_copy(x_hbm.at[base], rx1, isems[b1]).wait()

        @pl.when(i >= 1)
        def _():
            pltpu.make_async_copy(ou0, out_hbm.at[base], osems[b0]).wait()
            pltpu.make_async_copy(ou1, out_hbm.at[base], osems[b1]).wait()

        @plsc.parallel_loop(0, _NG, unroll=8)
        def _g(g):
            off = g * _L
            ipk = ipk_v[pl.ds(off, _L)]
            ia = lax.bitwise_and(ipk, jnp.int32(0xFFFF))
            ib = lax.shift_right_logical(ipk, jnp.int32(16))
            k0 = c0_v[pl.ds(off, _L)]
            ka = ca_v[pl.ds(off, _L)]
            kb = cb_v[pl.ds(off, _L)]
            kab = cab_v[pl.ds(off, _L)]
            a0 = plsc.load_gather(rx0, [ia])
            b0v = plsc.load_gather(rx0, [ib])
            a1 = plsc.load_gather(rx1, [ia])
            b1v = plsc.load_gather(rx1, [ib])
            ou0[pl.ds(off, _L)] = (k0 + ka * a0) + b0v * (kb + kab * a0)
            ou1[pl.ds(off, _L)] = (k0 + ka * a1) + b1v * (kb + kab * a1)

        pltpu.async_copy(ou0, out_hbm.at[base + r0], osems[b0])
        pltpu.async_copy(ou1, out_hbm.at[base + r0 + 1], osems[b1])

        @pl.when(i < _ROWS_PER_TILE // 4 - 1)
        def _():
            pltpu.async_copy(x_hbm.at[base + r0 + 4], rx0, isems[b0])
            pltpu.async_copy(x_hbm.at[base + r0 + 5], rx1, isems[b1])

    def _iter(i, carry):
        _pair(i, 0)
        _pair(i, 1)
        return carry

    lax.fori_loop(0, _ROWS_PER_TILE // 4, _iter, 0)

    for b in range(4):
        pltpu.make_async_copy(outs[b], out_hbm.at[base], osems[b]).wait()


_sc_main = functools.partial(
    pl.kernel,
    out_type=(jax.ShapeDtypeStruct((_B, _OUT), jnp.float32),
              jax.ShapeDtypeStruct((_NC, 4, _OUT), jnp.float32)),
    mesh=plsc.VectorSubcoreMesh(core_axis_name="c", subcore_axis_name="s"),
    compiler_params=pltpu.CompilerParams(needs_layout_passes=False),
    scratch_types=[
        pltpu.VMEM((_IN,), jnp.float32),
        pltpu.VMEM((_IN,), jnp.float32),
        pltpu.VMEM((_IN,), jnp.float32),
        pltpu.VMEM((_IN,), jnp.float32),
        pltpu.VMEM((_OUT,), jnp.float32),
        pltpu.VMEM((_OUT,), jnp.float32),
        pltpu.VMEM((_OUT,), jnp.float32),
        pltpu.VMEM((_OUT,), jnp.float32),
        pltpu.VMEM((_OUT,), jnp.int32),
        pltpu.VMEM((_OUT,), jnp.float32),
        pltpu.VMEM((_OUT,), jnp.float32),
        pltpu.VMEM((_OUT,), jnp.float32),
        pltpu.VMEM((_OUT,), jnp.float32),
        pltpu.VMEM((_JS * 16,), jnp.float32),
        pltpu.VMEM((_JS,), jnp.float32),
        pltpu.VMEM((_JS,), jnp.float32),
        pltpu.VMEM((_JS,), jnp.float32),
        pltpu.VMEM((_JS,), jnp.float32),
        pltpu.SemaphoreType.DMA,
        pltpu.SemaphoreType.DMA,
        pltpu.SemaphoreType.DMA,
        pltpu.SemaphoreType.DMA,
        pltpu.SemaphoreType.DMA,
        pltpu.SemaphoreType.DMA,
        pltpu.SemaphoreType.DMA,
        pltpu.SemaphoreType.DMA,
        pltpu.SemaphoreType.DMA,
    ],
)(_sc_body)


def kernel(x, weights, idx_a, idx_b):
    ipk = jnp.bitwise_or(idx_a.astype(jnp.int32),
                         jnp.left_shift(idx_b.astype(jnp.int32), 16))
    out, _ = _sc_main(x, weights.reshape(-1), ipk)
    return out


# speedup vs baseline: 4.9220x; 1.1687x over previous
import functools

import jax
import jax.numpy as jnp
from jax import lax
from jax.experimental import pallas as pl
from jax.experimental.pallas import tpu as pltpu
from jax.experimental.pallas import tpu_sc as plsc

_B = 2048
_IN = 8192
_OUT = 8192
_L = 16
_NC = 2
_NS = 16
_NW = _NC * _NS
_ROWS_PER_TILE = _B // _NW
_NG = _OUT // _L
_JS = _OUT // _NS


def _sc_body(x_hbm, w_hbm, ipk_hbm,
             out_hbm, cof_hbm,
             row0_v, row1_v, row2_v, row3_v,
             o0_v, o1_v, o2_v, o3_v,
             ipk_v, c01_v, c23_v,
             w_v, st0_v, sta_v,
             isem0, isem1, isem2, isem3, osem0, osem1, osem2, osem3,
             ipksem):
    c = lax.axis_index("c")
    s = lax.axis_index("s")
    wid = s * _NC + c
    base = wid * _ROWS_PER_TILE

    rows = (row0_v, row1_v, row2_v, row3_v)
    outs = (o0_v, o1_v, o2_v, o3_v)
    isems = (isem0, isem1, isem2, isem3)
    osems = (osem0, osem1, osem2, osem3)

    pltpu.async_copy(ipk_hbm, ipk_v, ipksem)
    for b in range(4):
        pltpu.async_copy(x_hbm.at[base + b], rows[b], isems[b])

    jbase = s * _JS
    pltpu.sync_copy(w_hbm.at[pl.ds(jbase * 16, _JS * 16)], w_v)

    def _cgroup(g, carry):
        j0 = g * _L
        stride = lax.iota(jnp.int32, _L) * 16
        cols = []
        for k in range(16):
            cols.append(plsc.load_gather(w_v, [j0 * 16 + k + stride]))
        m = cols[0]
        for k in range(1, 16):
            m = jnp.maximum(m, cols[k])
        e = [jnp.exp(col - m) for col in cols]
        tot = e[0]
        for k in range(1, 16):
            tot = tot + e[k]
        inv = 1.0 / tot
        c0 = (e[8] + e[9] + e[10] + e[11]
              + e[12] + e[13] + e[14] + e[15]) * inv
        ca = (e[2] + e[3] + e[6] + e[7]
              - e[8] - e[9] - e[12] - e[13]) * inv
        cb = (e[4] + e[5] + e[6] + e[7]
              - e[8] - e[9] - e[10] - e[11]) * inv
        cab = (e[1] - e[2] - e[4] - 2.0 * e[6] - e[7]
               + e[8] + 2.0 * e[9] + e[11] + e[13] - e[14]) * inv
        pk01 = plsc.pack(c0, ca, format=plsc.PackFormat.INTERLEAVED)
        pk23 = plsc.pack(cb, cab, format=plsc.PackFormat.INTERLEAVED)
        st0_v[pl.ds(j0, _L)] = plsc.bitcast(pk01, jnp.float32)
        sta_v[pl.ds(j0, _L)] = plsc.bitcast(pk23, jnp.float32)
        return carry

    lax.fori_loop(0, _JS // _L, _cgroup, 0)

    pltpu.sync_copy(st0_v, cof_hbm.at[c, 0, pl.ds(jbase, _JS)])
    pltpu.sync_copy(sta_v, cof_hbm.at[c, 1, pl.ds(jbase, _JS)])
    plsc.subcore_barrier()
    pltpu.sync_copy(cof_hbm.at[c, 0], c01_v)
    pltpu.sync_copy(cof_hbm.at[c, 1], c23_v)

    pltpu.make_async_copy(ipk_hbm, ipk_v, ipksem).wait()

    def _pair(i, h):
        b0, b1 = 2 * h, 2 * h + 1
        r0 = 4 * i + 2 * h
        rx0, rx1 = rows[b0], rows[b1]
        ou0, ou1 = outs[b0], outs[b1]
        pltpu.make_async_copy(x_hbm.at[base], rx0, isems[b0]).wait()
        pltpu.make_async_copy(x_hbm.at[base], rx1, isems[b1]).wait()

        @pl.when(i >= 1)
        def _():
            pltpu.make_async_copy(ou0, out_hbm.at[base], osems[b0]).wait()
            pltpu.make_async_copy(ou1, out_hbm.at[base], osems[b1]).wait()

        @plsc.parallel_loop(0, _NG, unroll=8)
        def _g(g):
            off = g * _L
            ipk = ipk_v[pl.ds(off, _L)]
            ia = lax.bitwise_and(ipk, jnp.int32(0xFFFF))
            ib = lax.shift_right_logical(ipk, jnp.int32(16))
            pk01 = plsc.bitcast(c01_v[pl.ds(off, _L)], jnp.bfloat16)
            pk23 = plsc.bitcast(c23_v[pl.ds(off, _L)], jnp.bfloat16)
            k0, ka = plsc.unpack(pk01, format=plsc.PackFormat.INTERLEAVED)
            kb, kab = plsc.unpack(pk23, format=plsc.PackFormat.INTERLEAVED)
            a0 = plsc.load_gather(rx0, [ia])
            b0v = plsc.load_gather(rx0, [ib])
            a1 = plsc.load_gather(rx1, [ia])
            b1v = plsc.load_gather(rx1, [ib])
            ou0[pl.ds(off, _L)] = (k0 + ka * a0) + b0v * (kb + kab * a0)
            ou1[pl.ds(off, _L)] = (k0 + ka * a1) + b1v * (kb + kab * a1)

        pltpu.async_copy(ou0, out_hbm.at[base + r0], osems[b0])
        pltpu.async_copy(ou1, out_hbm.at[base + r0 + 1], osems[b1])

        @pl.when(i < _ROWS_PER_TILE // 4 - 1)
        def _():
            pltpu.async_copy(x_hbm.at[base + r0 + 4], rx0, isems[b0])
            pltpu.async_copy(x_hbm.at[base + r0 + 5], rx1, isems[b1])

    def _iter(i, carry):
        _pair(i, 0)
        _pair(i, 1)
        return carry

    lax.fori_loop(0, _ROWS_PER_TILE // 4, _iter, 0)

    for b in range(4):
        pltpu.make_async_copy(outs[b], out_hbm.at[base], osems[b]).wait()


_sc_main = functools.partial(
    pl.kernel,
    out_type=(jax.ShapeDtypeStruct((_B, _OUT), jnp.float32),
              jax.ShapeDtypeStruct((_NC, 2, _OUT), jnp.float32)),
    mesh=plsc.VectorSubcoreMesh(core_axis_name="c", subcore_axis_name="s"),
    compiler_params=pltpu.CompilerParams(needs_layout_passes=False),
    scratch_types=[
        pltpu.VMEM((_IN,), jnp.float32),
        pltpu.VMEM((_IN,), jnp.float32),
        pltpu.VMEM((_IN,), jnp.float32),
        pltpu.VMEM((_IN,), jnp.float32),
        pltpu.VMEM((_OUT,), jnp.float32),
        pltpu.VMEM((_OUT,), jnp.float32),
        pltpu.VMEM((_OUT,), jnp.float32),
        pltpu.VMEM((_OUT,), jnp.float32),
        pltpu.VMEM((_OUT,), jnp.int32),
        pltpu.VMEM((_OUT,), jnp.float32),
        pltpu.VMEM((_OUT,), jnp.float32),
        pltpu.VMEM((_JS * 16,), jnp.float32),
        pltpu.VMEM((_JS,), jnp.float32),
        pltpu.VMEM((_JS,), jnp.float32),
        pltpu.SemaphoreType.DMA,
        pltpu.SemaphoreType.DMA,
        pltpu.SemaphoreType.DMA,
        pltpu.SemaphoreType.DMA,
        pltpu.SemaphoreType.DMA,
        pltpu.SemaphoreType.DMA,
        pltpu.SemaphoreType.DMA,
        pltpu.SemaphoreType.DMA,
        pltpu.SemaphoreType.DMA,
    ],
)(_sc_body)


def kernel(x, weights, idx_a, idx_b):
    ipk = jnp.bitwise_or(idx_a.astype(jnp.int32),
                         jnp.left_shift(idx_b.astype(jnp.int32), 16))
    out, _ = _sc_main(x, weights.reshape(-1), ipk)
    return out
